# baseline (speedup 1.0000x reference)
import functools

import jax
import jax.numpy as jnp
from jax import lax
from jax.experimental import pallas as pl
from jax.experimental.pallas import tpu as pltpu
from jax.experimental.pallas import tpu_sc as plsc

L = 16


def _rsqrt(v):
    i = lax.bitcast_convert_type(v, jnp.int32)
    i = jnp.full((L,), 0x5F3759DF, jnp.int32) - lax.shift_right_logical(i, 1)
    y = lax.bitcast_convert_type(i, jnp.float32)
    half = v * 0.5
    for _ in range(2):
        y = y * (1.5 - half * y * y)
    return y


def _make_kernel(B, S, V, D, NC, NS):
    NW = NC * NS
    BW = B // NW
    assert BW == 128 and D == 64 and NS == 16
    KD = D // L

    mesh = plsc.VectorSubcoreMesh(core_axis_name="c", subcore_axis_name="s")

    @functools.partial(
        pl.kernel,
        mesh=mesh,
        compiler_params=pltpu.CompilerParams(use_tc_tiling_on_sc=False,
                                             needs_layout_passes=False),
        out_type=jax.ShapeDtypeStruct((S, 8, NW, 8, 128), jnp.float32),
        scratch_types=[
            pltpu.VMEM((S, BW), jnp.int32),
            pltpu.VMEM((BW, 128), jnp.float32),
            pltpu.VMEM((BW, 128), jnp.float32),
            pltpu.VMEM((8, 8, 128), jnp.float32),
            pltpu.VMEM_SHARED((2, 8, NS, 8, 128), jnp.float32),
            pltpu.VMEM((S, D), jnp.float32),
            pltpu.VMEM((D,), jnp.float32),
            pltpu.VMEM((D,), jnp.float32),
            pltpu.SemaphoreType.DMA,
            pltpu.SemaphoreType.DMA,
            pltpu.SemaphoreType.DMA,
        ],
    )
    def k(xg_hbm, table_hbm, gamma_hbm, beta_hbm, pos_hbm, out_hbm,
          idx_v, rows0, rows1, stage, slab, pos_v, gam_v, bet_v,
          gsem0, gsem1, ssem):
        cid = lax.axis_index("c")
        tid = lax.axis_index("s")
        wid = cid * NS + tid

        pltpu.sync_copy(xg_hbm.at[wid], idx_v)
        pltpu.sync_copy(pos_hbm, pos_v)
        pltpu.sync_copy(gamma_hbm, gam_v)
        pltpu.sync_copy(beta_hbm, bet_v)

        gvs = [gam_v[pl.ds(L * t, L)] for t in range(KD)]
        bvs = [bet_v[pl.ds(L * t, L)] for t in range(KD)]
        inv_d = jnp.float32(1.0 / D)

        lane_ids = lax.iota(jnp.int32, L)
        _dnums = lax.GatherDimensionNumbers(
            offset_dims=(), collapsed_slice_dims=(0,), start_index_map=(0,))

        def P(v, idx):
            return lax.gather(v, idx, _dnums, slice_sizes=(1,),
                              unique_indices=True,
                              mode=lax.GatherScatterMode.PROMISE_IN_BOUNDS)

        perm8i, perm4i, perm2i, perm1i = (
            jnp.reshape(lane_ids ^ sh, (L, 1)) for sh in (8, 4, 2, 1))
        m8 = lane_ids < 8
        m4 = (lane_ids & 4) == 0
        lo2 = lane_ids & 3
        bidx = [jnp.reshape(lo2 + off, (L, 1)) for off in (0, 8, 4, 12)]

        def pack4(x0, x1, x2, x3):
            t0, t1, t2, t3 = (x + P(x, perm8i) for x in (x0, x1, x2, x3))
            u01 = jnp.where(m8, t0, t1)
            u23 = jnp.where(m8, t2, t3)
            v01 = u01 + P(u01, perm4i)
            v23 = u23 + P(u23, perm4i)
            w = jnp.where(m4, v01, v23)
            w = w + P(w, perm2i)
            return w + P(w, perm1i)

        fg_lo = lax.shift_right_logical(lane_ids, 3)
        f_idx = lane_ids & 7
        sfg = [fg_lo + (2 * t) for t in range(KD)]

        def start_gather(si, rows, gsem):
            pltpu.async_copy(table_hbm.at[idx_v.at[si]], rows, gsem)

        def wait_gather(rows, gsem):
            pltpu.make_async_copy(table_hbm.at[pl.ds(0, BW)], rows,
                                  gsem).wait()

        def drain_stores():
            for fg in range(8):
                pltpu.make_async_copy(slab.at[0, fg],
                                      out_hbm.at[0, fg, pl.ds(0, NS)],
                                      ssem).wait()

        start_gather(0, rows0, gsem0)

        def do_chunk(si, rows, gsem, n_rows, n_gsem):
            wait_gather(rows, gsem)
            @pl.when(si + 1 < S)
            def _():
                start_gather(si + 1, n_rows, n_gsem)

            pv = [pos_v[si, pl.ds(L * t, L)] for t in range(KD)]

            def blk_body(jj, _):
                j = jj * 4
                hs = []
                ss = []
                qs = []
                for r in range(4):
                    h = [rows[j + r, pl.ds(L * t, L)] + pv[t]
                         for t in range(KD)]
                    hs.append(h)
                    ss.append((h[0] + h[1]) + (h[2] + h[3]))
                    qs.append((h[0] * h[0] + h[1] * h[1])
                              + (h[2] * h[2] + h[3] * h[3]))
                s4 = pack4(*ss)
                q4 = pack4(*qs)
                mean4 = s4 * inv_d
                var4 = q4 * inv_d - mean4 * mean4 + 1e-5
                inv4 = _rsqrt(var4)
                for r in range(4):
                    mean_r = P(mean4, bidx[r])
                    inv_r = P(inv4, bidx[r])
                    col = jnp.broadcast_to(j + r, (L,)).astype(jnp.int32)
                    for t in range(KD):
                        y = (hs[r][t] - mean_r) * inv_r * gvs[t] + bvs[t]
                        plsc.store_scatter(stage, [sfg[t], f_idx, col], y)
                return 0

            lax.fori_loop(0, BW // 4, blk_body, 0, unroll=2)

            buf = lax.rem(si, 2)
            for fg in range(8):
                pltpu.sync_copy(stage.at[fg], slab.at[buf, fg, tid])
            @pl.when(jnp.logical_and(si >= 1, tid == lax.rem(si - 1, NS)))
            def _():
                drain_stores()
            plsc.subcore_barrier()
            @pl.when(tid == lax.rem(si, NS))
            def _():
                for fg in range(8):
                    pltpu.async_copy(
                        slab.at[buf, fg],
                        out_hbm.at[si, fg, pl.ds(cid * NS, NS)], ssem)

        def outer(go, _):
            for b in range(2):
                si = go * 2 + b
                if b == 0:
                    do_chunk(si, rows0, gsem0, rows1, gsem1)
                else:
                    do_chunk(si, rows1, gsem1, rows0, gsem0)
            return 0

        lax.fori_loop(0, S // 2, outer, 0)
        @pl.when(tid == lax.rem(S - 1, NS))
        def _():
            drain_stores()

    return k


def kernel(x, tok_table, gamma, beta, pos_embed):
    B, S = x.shape
    V, D = tok_table.shape
    info = plsc.get_sparse_core_info()
    NC, NS = info.num_cores, info.num_subcores
    NW = NC * NS
    BW = B // NW
    k = _make_kernel(B, S, V, D, NC, NS)
    xg = x.T.reshape(S, NW, BW).transpose(1, 0, 2)
    table128 = jnp.pad(tok_table, ((0, 0), (0, 128 - D)))
    out = k(xg, table128, gamma, beta, pos_embed)
    y = out.transpose(2, 4, 0, 1, 3)
    return y.reshape(B, S, D)


# speedup vs baseline: 1.3297x; 1.3297x over previous
import functools

import jax
import jax.numpy as jnp
from jax import lax
from jax.experimental import pallas as pl
from jax.experimental.pallas import tpu as pltpu
from jax.experimental.pallas import tpu_sc as plsc

L = 16


def _rsqrt(v):
    i = lax.bitcast_convert_type(v, jnp.int32)
    i = jnp.full((L,), 0x5F3759DF, jnp.int32) - lax.shift_right_logical(i, 1)
    y = lax.bitcast_convert_type(i, jnp.float32)
    half = v * 0.5
    for _ in range(2):
        y = y * (1.5 - half * y * y)
    return y


def _make_kernel(B, S, V, D, NC, NS):
    NW = NC * NS
    N = B * S
    CHUNK = 128
    per_w = N // NW
    n_chunks = per_w // CHUNK
    assert N % NW == 0 and per_w % CHUNK == 0 and D % L == 0
    KD = D // L

    mesh = plsc.VectorSubcoreMesh(core_axis_name="c", subcore_axis_name="s")

    @functools.partial(
        pl.kernel,
        mesh=mesh,
        compiler_params=pltpu.CompilerParams(use_tc_tiling_on_sc=False),
        out_type=jax.ShapeDtypeStruct((N // 2, 2 * D), jnp.float32),
        scratch_types=[
            pltpu.VMEM((n_chunks, CHUNK), jnp.int32),
            pltpu.VMEM((CHUNK, 2 * D), jnp.float32),
            pltpu.VMEM((CHUNK, 2 * D), jnp.float32),
            pltpu.VMEM((CHUNK // 2, 2 * D), jnp.float32),
            pltpu.VMEM((CHUNK // 2, 2 * D), jnp.float32),
            pltpu.VMEM((S, D), jnp.float32),
            pltpu.VMEM((D,), jnp.float32),
            pltpu.VMEM((D,), jnp.float32),
            pltpu.SemaphoreType.DMA,
            pltpu.SemaphoreType.DMA,
            pltpu.SemaphoreType.DMA,
            pltpu.SemaphoreType.DMA,
        ],
    )
    def k(x_hbm, table_hbm, gamma_hbm, beta_hbm, pos_hbm, out_hbm,
          idx_v, rows0, rows1, st0, st1, pos_v, gam_v, bet_v,
          gsem0, gsem1, ssem0, ssem1):
        wid = lax.axis_index("s") * NC + lax.axis_index("c")
        wbase = wid * per_w

        pltpu.sync_copy(x_hbm.at[wid], idx_v)
        pltpu.sync_copy(pos_hbm, pos_v)
        pltpu.sync_copy(gamma_hbm, gam_v)
        pltpu.sync_copy(beta_hbm, bet_v)

        gvs = [gam_v[pl.ds(L * t, L)] for t in range(KD)]
        bvs = [bet_v[pl.ds(L * t, L)] for t in range(KD)]
        inv_d = jnp.float32(1.0 / D)

        lane_ids = lax.iota(jnp.int32, L)
        _dnums = lax.GatherDimensionNumbers(
            offset_dims=(), collapsed_slice_dims=(0,), start_index_map=(0,))

        def P(v, idx):
            return lax.gather(v, idx, _dnums, slice_sizes=(1,),
                              unique_indices=True,
                              mode=lax.GatherScatterMode.PROMISE_IN_BOUNDS)

        perm8i, perm4i, perm2i, perm1i = (
            jnp.reshape(lane_ids ^ sh, (L, 1)) for sh in (8, 4, 2, 1))
        m8 = lane_ids < 8
        m4 = (lane_ids & 4) == 0
        lo2 = lane_ids & 3
        bidx = [jnp.reshape(lo2 + off, (L, 1)) for off in (0, 8, 4, 12)]

        def pack4(x0, x1, x2, x3):
            t0, t1, t2, t3 = (x + P(x, perm8i) for x in (x0, x1, x2, x3))
            u01 = jnp.where(m8, t0, t1)
            u23 = jnp.where(m8, t2, t3)
            v01 = u01 + P(u01, perm4i)
            v23 = u23 + P(u23, perm4i)
            w = jnp.where(m4, v01, v23)
            w = w + P(w, perm2i)
            return w + P(w, perm1i)

        def start_gather(c, rows, gsem):
            pltpu.async_copy(table_hbm.at[idx_v.at[c]], rows, gsem)

        def wait_gather(rows, gsem):
            pltpu.make_async_copy(table_hbm.at[pl.ds(0, CHUNK)], rows,
                                  gsem).wait()

        def wait_store(st, ssem):
            pltpu.make_async_copy(st, out_hbm.at[pl.ds(0, CHUNK // 2)],
                                  ssem).wait()

        start_gather(0, rows0, gsem0)

        def do_chunk(c, rows, gsem, st, ssem, n_rows, n_gsem):
            base = wbase + c * CHUNK
            wait_gather(rows, gsem)
            @pl.when(c + 1 < n_chunks)
            def _():
                start_gather(c + 1, n_rows, n_gsem)
            @pl.when(c >= 2)
            def _():
                wait_store(st, ssem)

            p0 = lax.rem(base, S)

            def blk_body(jj, _):
                j = jj * 4
                pj = p0 + j
                hs = []
                ss = []
                qs = []
                for r in range(4):
                    pr = pj + r
                    pr = jnp.where(pr < S, pr, pr - S)
                    h = [rows[j + r, pl.ds(L * t, L)]
                         + pos_v[pr, pl.ds(L * t, L)] for t in range(KD)]
                    hs.append(h)
                    ss.append((h[0] + h[1]) + (h[2] + h[3]))
                    qs.append((h[0] * h[0] + h[1] * h[1])
                              + (h[2] * h[2] + h[3] * h[3]))
                s4 = pack4(*ss)
                q4 = pack4(*qs)
                mean4 = s4 * inv_d
                var4 = q4 * inv_d - mean4 * mean4 + 1e-5
                inv4 = _rsqrt(var4)
                for r in range(4):
                    mean_r = P(mean4, bidx[r])
                    inv_r = P(inv4, bidx[r])
                    for t in range(KD):
                        st[jj * 2 + r // 2,
                           pl.ds(D * (r % 2) + L * t, L)] = (
                            (hs[r][t] - mean_r) * inv_r * gvs[t] + bvs[t])
                return 0

            lax.fori_loop(0, CHUNK // 4, blk_body, 0, unroll=2)
            pltpu.async_copy(st, out_hbm.at[pl.ds(base // 2, CHUNK // 2)],
                             ssem)

        def outer(go, _):
            for b in range(2):
                c = go * 2 + b
                if b == 0:
                    do_chunk(c, rows0, gsem0, st0, ssem0, rows1, gsem1)
                else:
                    do_chunk(c, rows1, gsem1, st1, ssem1, rows0, gsem0)
            return 0

        lax.fori_loop(0, n_chunks // 2, outer, 0)
        wait_store(st0, ssem0)
        wait_store(st1, ssem1)

    return k


def kernel(x, tok_table, gamma, beta, pos_embed):
    B, S = x.shape
    V, D = tok_table.shape
    info = plsc.get_sparse_core_info()
    NC, NS = info.num_cores, info.num_subcores
    NW = NC * NS
    N = B * S
    CHUNK = 128
    per_w = N // NW
    k = _make_kernel(B, S, V, D, NC, NS)
    x_resh = x.reshape(NW, per_w // CHUNK, CHUNK)
    table128 = jnp.pad(tok_table, ((0, 0), (0, 128 - D)))
    out = k(x_resh, table128, gamma, beta, pos_embed)
    return out.reshape(B, S, D)
